# Initial kernel scaffold; baseline (speedup 1.0000x reference)
import jax
import jax.numpy as jnp
from jax.experimental import pallas as pl


def kernel(input_ids, category_ids, position_ids, id_table, cat_table, pos_table, W, b, ln_w, ln_b):
    raise NotImplementedError("write your pallas kernel here")


# speedup vs baseline: 3.4469x; 3.4469x over previous
import functools

import jax
import jax.numpy as jnp
from jax import lax
from jax.experimental import pallas as pl
from jax.experimental.pallas import tpu as pltpu
from jax.experimental.pallas import tpu_sc as plsc

VOCAB = 100000
CAT = 1000
MAXPOS = 200
EMB = 128
HID = 512
B = 1024
S = 200
N = B * S
EPS = 1e-12

NC = 2
NS = 16
NW = NC * NS
CHUNK = 128
TOK_PER_W = N // NW
N_CHUNKS = TOK_PER_W // CHUNK

TB = 512


def _sc_gather_body(ids_hbm, cats_hbm, poss_hbm, id_tab, cat_tab, pos_tab,
                    o1, o2, o3, idx1, idx2, idx3, r1, r2, r3, sem):
    wid = lax.axis_index("s") * NC + lax.axis_index("c")
    wbase = wid * TOK_PER_W

    def chunk(i, _):
        base = wbase + i * CHUNK
        pltpu.sync_copy(ids_hbm.at[pl.ds(base, CHUNK)], idx1)
        pltpu.sync_copy(cats_hbm.at[pl.ds(base, CHUNK)], idx2)
        pltpu.sync_copy(poss_hbm.at[pl.ds(base, CHUNK)], idx3)
        c1 = pltpu.async_copy(id_tab.at[idx1], r1, sem)
        c2 = pltpu.async_copy(cat_tab.at[idx2], r2, sem)
        c3 = pltpu.async_copy(pos_tab.at[idx3], r3, sem)
        c1.wait()
        c2.wait()
        c3.wait()
        pltpu.sync_copy(r1, o1.at[pl.ds(base, CHUNK)])
        pltpu.sync_copy(r2, o2.at[pl.ds(base, CHUNK)])
        pltpu.sync_copy(r3, o3.at[pl.ds(base, CHUNK)])
        return _

    lax.fori_loop(0, N_CHUNKS, chunk, None)


def _sc_gather(ids, cats, poss, id_tab, cat_tab, pos_tab):
    mesh = plsc.VectorSubcoreMesh(core_axis_name="c", subcore_axis_name="s")
    row = jax.ShapeDtypeStruct((N, EMB), jnp.float32)
    f = pl.kernel(
        _sc_gather_body,
        out_type=(row, row, row),
        mesh=mesh,
        scratch_types=[
            pltpu.VMEM((CHUNK,), jnp.int32),
            pltpu.VMEM((CHUNK,), jnp.int32),
            pltpu.VMEM((CHUNK,), jnp.int32),
            pltpu.VMEM((CHUNK, EMB), jnp.float32),
            pltpu.VMEM((CHUNK, EMB), jnp.float32),
            pltpu.VMEM((CHUNK, EMB), jnp.float32),
            pltpu.SemaphoreType.DMA,
        ],
    )
    return f(ids, cats, poss, id_tab, cat_tab, pos_tab)


def _tc_body(x1, x2, x3, w1, w2, w3, bb, lw, lb, o):
    acc = jnp.dot(x1[...], w1[...], preferred_element_type=jnp.float32)
    acc += jnp.dot(x2[...], w2[...], preferred_element_type=jnp.float32)
    acc += jnp.dot(x3[...], w3[...], preferred_element_type=jnp.float32)
    acc += bb[...]
    m = jnp.mean(acc, axis=-1, keepdims=True)
    d = acc - m
    v = jnp.mean(d * d, axis=-1, keepdims=True)
    o[...] = d * lax.rsqrt(v + EPS) * lw[...] + lb[...]


def _tc_linear_ln(x1, x2, x3, Wt, b, ln_w, ln_b):
    grid = (N // TB,)
    xspec = pl.BlockSpec((TB, EMB), lambda i: (i, 0))
    wspec = pl.BlockSpec((EMB, HID), lambda i: (0, 0))
    vspec = pl.BlockSpec((1, HID), lambda i: (0, 0))
    return pl.pallas_call(
        _tc_body,
        grid=grid,
        in_specs=[xspec, xspec, xspec, wspec, wspec, wspec, vspec, vspec, vspec],
        out_specs=pl.BlockSpec((TB, HID), lambda i: (i, 0)),
        out_shape=jax.ShapeDtypeStruct((N, HID), jnp.float32),
    )(x1, x2, x3, Wt[:EMB], Wt[EMB:2 * EMB], Wt[2 * EMB:], b.reshape(1, HID),
      ln_w.reshape(1, HID), ln_b.reshape(1, HID))


def kernel(input_ids, category_ids, position_ids, id_table, cat_table,
           pos_table, W, b, ln_w, ln_b):
    ids = input_ids.reshape(-1).astype(jnp.int32)
    cats = category_ids.reshape(-1).astype(jnp.int32)
    poss = position_ids.reshape(-1).astype(jnp.int32)
    e1, e2, e3 = _sc_gather(ids, cats, poss, id_table, cat_table, pos_table)
    out = _tc_linear_ln(e1, e2, e3, W.T, b, ln_w, ln_b)
    return out.reshape(B, S, HID)


# speedup vs baseline: 3.5314x; 1.0245x over previous
import functools

import jax
import jax.numpy as jnp
from jax import lax
from jax.experimental import pallas as pl
from jax.experimental.pallas import tpu as pltpu
from jax.experimental.pallas import tpu_sc as plsc

VOCAB = 100000
CAT = 1000
MAXPOS = 200
EMB = 128
HID = 512
B = 1024
S = 200
N = B * S
EPS = 1e-12

NC = 2
NS = 16
NW = NC * NS
CHUNK = 128
TOK_PER_W = N // NW
N_CHUNKS = TOK_PER_W // CHUNK

TB = 512


def _sc_gather_body(ids_hbm, cats_hbm, poss_hbm, id_tab, cat_tab, pos_tab,
                    o1, o2, o3, idx1, idx2, idx3,
                    r1a, r1b, r2a, r2b, r3a, r3b, sem0, sem1):
    wid = lax.axis_index("s") * NC + lax.axis_index("c")
    wbase = wid * TOK_PER_W
    tabs = (id_tab, cat_tab, pos_tab)
    idxs = (idx1, idx2, idx3)
    outs = (o1, o2, o3)
    bufs = ((r1a, r1b), (r2a, r2b), (r3a, r3b))
    sems = (sem0, sem1)

    pltpu.sync_copy(ids_hbm.at[pl.ds(wbase, TOK_PER_W)], idx1)
    pltpu.sync_copy(cats_hbm.at[pl.ds(wbase, TOK_PER_W)], idx2)
    pltpu.sync_copy(poss_hbm.at[pl.ds(wbase, TOK_PER_W)], idx3)

    def fire(c, s):
        off = c * CHUNK
        for t in range(3):
            pltpu.async_copy(tabs[t].at[idxs[t].at[pl.ds(off, CHUNK)]],
                             bufs[t][s], sems[s])

    def drain(s):
        for t in range(3):
            pltpu.make_async_copy(o1.at[pl.ds(0, CHUNK)], bufs[t][s],
                                  sems[s]).wait()

    def writeback(c, s):
        base = wbase + c * CHUNK
        for t in range(3):
            pltpu.sync_copy(bufs[t][s], outs[t].at[pl.ds(base, CHUNK)])

    fire(0, 0)
    fire(1, 1)

    def pair(p, _):
        c = 2 * p
        drain(0)
        writeback(c, 0)
        fire(c + 2, 0)
        drain(1)
        writeback(c + 1, 1)
        fire(c + 3, 1)
        return _

    lax.fori_loop(0, N_CHUNKS // 2 - 1, pair, None)
    drain(0)
    writeback(N_CHUNKS - 2, 0)
    drain(1)
    writeback(N_CHUNKS - 1, 1)


def _sc_gather(ids, cats, poss, id_tab, cat_tab, pos_tab):
    mesh = plsc.VectorSubcoreMesh(core_axis_name="c", subcore_axis_name="s")
    row = jax.ShapeDtypeStruct((N, EMB), jnp.float32)
    f = pl.kernel(
        _sc_gather_body,
        out_type=(row, row, row),
        mesh=mesh,
        scratch_types=[
            pltpu.VMEM((TOK_PER_W,), jnp.int32),
            pltpu.VMEM((TOK_PER_W,), jnp.int32),
            pltpu.VMEM((TOK_PER_W,), jnp.int32),
            pltpu.VMEM((CHUNK, EMB), jnp.float32),
            pltpu.VMEM((CHUNK, EMB), jnp.float32),
            pltpu.VMEM((CHUNK, EMB), jnp.float32),
            pltpu.VMEM((CHUNK, EMB), jnp.float32),
            pltpu.VMEM((CHUNK, EMB), jnp.float32),
            pltpu.VMEM((CHUNK, EMB), jnp.float32),
            pltpu.SemaphoreType.DMA,
            pltpu.SemaphoreType.DMA,
        ],
    )
    return f(ids, cats, poss, id_tab, cat_tab, pos_tab)


def _tc_body(x1, x2, x3, w1, w2, w3, bb, lw, lb, o):
    acc = jnp.dot(x1[...], w1[...], preferred_element_type=jnp.float32)
    acc += jnp.dot(x2[...], w2[...], preferred_element_type=jnp.float32)
    acc += jnp.dot(x3[...], w3[...], preferred_element_type=jnp.float32)
    acc += bb[...]
    m = jnp.mean(acc, axis=-1, keepdims=True)
    d = acc - m
    v = jnp.mean(d * d, axis=-1, keepdims=True)
    o[...] = d * lax.rsqrt(v + EPS) * lw[...] + lb[...]


def _tc_linear_ln(x1, x2, x3, Wt, b, ln_w, ln_b):
    grid = (N // TB,)
    xspec = pl.BlockSpec((TB, EMB), lambda i: (i, 0))
    wspec = pl.BlockSpec((EMB, HID), lambda i: (0, 0))
    vspec = pl.BlockSpec((1, HID), lambda i: (0, 0))
    return pl.pallas_call(
        _tc_body,
        grid=grid,
        in_specs=[xspec, xspec, xspec, wspec, wspec, wspec, vspec, vspec, vspec],
        out_specs=pl.BlockSpec((TB, HID), lambda i: (i, 0)),
        out_shape=jax.ShapeDtypeStruct((N, HID), jnp.float32),
    )(x1, x2, x3, Wt[:EMB], Wt[EMB:2 * EMB], Wt[2 * EMB:], b.reshape(1, HID),
      ln_w.reshape(1, HID), ln_b.reshape(1, HID))


def kernel(input_ids, category_ids, position_ids, id_table, cat_table,
           pos_table, W, b, ln_w, ln_b):
    ids = input_ids.reshape(-1).astype(jnp.int32)
    cats = category_ids.reshape(-1).astype(jnp.int32)
    poss = position_ids.reshape(-1).astype(jnp.int32)
    e1, e2, e3 = _sc_gather(ids, cats, poss, id_table, cat_table, pos_table)
    out = _tc_linear_ln(e1, e2, e3, W.T, b, ln_w, ln_b)
    return out.reshape(B, S, HID)


# speedup vs baseline: 3.7201x; 1.0534x over previous
import jax
import jax.numpy as jnp
from jax import lax
from jax.experimental import pallas as pl
from jax.experimental.pallas import tpu as pltpu
from jax.experimental.pallas import tpu_sc as plsc

VOCAB = 100000
CAT = 1000
MAXPOS = 200
EMB = 128
HID = 512
B = 1024
S = 200
N = B * S
EPS = 1e-12

NC = 2
NS = 16
NW = NC * NS
CHUNK = 128
TOK_PER_W = N // NW
N_CHUNKS = TOK_PER_W // CHUNK

TB = 512


def _sc_gather_body(ids_hbm, id_tab, o1, idx1, r0, r1, sem0, sem1):
    wid = lax.axis_index("s") * NC + lax.axis_index("c")
    wbase = wid * TOK_PER_W
    bufs = (r0, r1)
    sems = (sem0, sem1)

    pltpu.sync_copy(ids_hbm.at[pl.ds(wbase, TOK_PER_W)], idx1)

    def fire(c, s):
        pltpu.async_copy(id_tab.at[idx1.at[pl.ds(c * CHUNK, CHUNK)]],
                         bufs[s], sems[s])

    def drain(s):
        pltpu.make_async_copy(o1.at[pl.ds(0, CHUNK)], bufs[s], sems[s]).wait()

    def writeback(c, s):
        pltpu.sync_copy(bufs[s], o1.at[pl.ds(wbase + c * CHUNK, CHUNK)])

    fire(0, 0)
    fire(1, 1)

    def pair(p, _):
        c = 2 * p
        drain(0)
        writeback(c, 0)
        fire(c + 2, 0)
        drain(1)
        writeback(c + 1, 1)
        fire(c + 3, 1)
        return _

    lax.fori_loop(0, N_CHUNKS // 2 - 1, pair, None)
    drain(0)
    writeback(N_CHUNKS - 2, 0)
    drain(1)
    writeback(N_CHUNKS - 1, 1)


def _sc_gather(ids, id_tab):
    mesh = plsc.VectorSubcoreMesh(core_axis_name="c", subcore_axis_name="s")
    f = pl.kernel(
        _sc_gather_body,
        out_type=jax.ShapeDtypeStruct((N, EMB), jnp.float32),
        mesh=mesh,
        scratch_types=[
            pltpu.VMEM((TOK_PER_W,), jnp.int32),
            pltpu.VMEM((CHUNK, EMB), jnp.float32),
            pltpu.VMEM((CHUNK, EMB), jnp.float32),
            pltpu.SemaphoreType.DMA,
            pltpu.SemaphoreType.DMA,
        ],
    )
    return f(ids, id_tab)


def _tc_body(x1, catb, posb, ct, pt, w1, w2, w3, bb, lw, lb, o):
    cat = catb[0, 0, :].reshape(TB, 1)
    pos = posb[0, 0, :].reshape(TB, 1)
    oh_c = (lax.broadcasted_iota(jnp.int32, (TB, CAT), 1) == cat).astype(jnp.bfloat16)
    oh_p = (lax.broadcasted_iota(jnp.int32, (TB, MAXPOS), 1) == pos).astype(jnp.bfloat16)
    x2 = jnp.dot(oh_c, ct[...], preferred_element_type=jnp.float32)
    x3 = jnp.dot(oh_p, pt[...], preferred_element_type=jnp.float32)
    acc = jnp.dot(x1[...], w1[...], preferred_element_type=jnp.float32)
    acc += jnp.dot(x2, w2[...], preferred_element_type=jnp.float32)
    acc += jnp.dot(x3, w3[...], preferred_element_type=jnp.float32)
    acc += bb[...]
    m = jnp.mean(acc, axis=-1, keepdims=True)
    d = acc - m
    v = jnp.mean(d * d, axis=-1, keepdims=True)
    o[...] = d * lax.rsqrt(v + EPS) * lw[...] + lb[...]


def _tc_linear_ln(x1, cats, poss, cat_tab, pos_tab, Wt, b, ln_w, ln_b):
    grid = (N // TB,)
    xspec = pl.BlockSpec((TB, EMB), lambda i: (i, 0))
    ispec = pl.BlockSpec((1, 1, TB), lambda i: (i, 0, 0))
    wspec = pl.BlockSpec((EMB, HID), lambda i: (0, 0))
    vspec = pl.BlockSpec((1, HID), lambda i: (0, 0))
    return pl.pallas_call(
        _tc_body,
        grid=grid,
        in_specs=[
            xspec, ispec, ispec,
            pl.BlockSpec((CAT, EMB), lambda i: (0, 0)),
            pl.BlockSpec((MAXPOS, EMB), lambda i: (0, 0)),
            wspec, wspec, wspec, vspec, vspec, vspec,
        ],
        out_specs=pl.BlockSpec((TB, HID), lambda i: (i, 0)),
        out_shape=jax.ShapeDtypeStruct((N, HID), jnp.float32),
    )(x1, cats.reshape(N // TB, 1, TB), poss.reshape(N // TB, 1, TB),
      cat_tab.astype(jnp.bfloat16), pos_tab.astype(jnp.bfloat16),
      Wt[:EMB], Wt[EMB:2 * EMB], Wt[2 * EMB:], b.reshape(1, HID),
      ln_w.reshape(1, HID), ln_b.reshape(1, HID))


def kernel(input_ids, category_ids, position_ids, id_table, cat_table,
           pos_table, W, b, ln_w, ln_b):
    ids = input_ids.reshape(-1).astype(jnp.int32)
    cats = category_ids.reshape(-1).astype(jnp.int32)
    poss = position_ids.reshape(-1).astype(jnp.int32)
    e1 = _sc_gather(ids, id_table)
    out = _tc_linear_ln(e1, cats, poss, cat_table, pos_table, W.T, b, ln_w, ln_b)
    return out.reshape(B, S, HID)


# speedup vs baseline: 4.1093x; 1.1046x over previous
import jax
import jax.numpy as jnp
from jax import lax
from jax.experimental import pallas as pl
from jax.experimental.pallas import tpu as pltpu
from jax.experimental.pallas import tpu_sc as plsc

VOCAB = 100000
CAT = 1000
MAXPOS = 200
EMB = 128
HID = 512
B = 1024
S = 200
N = B * S
EPS = 1e-12

NC = 2
NS = 16
NW = NC * NS
CHUNK = 128
TOK_PER_W = N // NW
N_CHUNKS = TOK_PER_W // CHUNK

TB = 1024


def _sc_gather_body(ids_hbm, id_tab, o1, idx1, r0, r1, sem0, sem1):
    wid = lax.axis_index("s") * NC + lax.axis_index("c")
    wbase = wid * TOK_PER_W
    bufs = (r0, r1)
    sems = (sem0, sem1)

    pltpu.sync_copy(ids_hbm.at[pl.ds(wbase, TOK_PER_W)], idx1)

    def fire(c, s):
        pltpu.async_copy(id_tab.at[idx1.at[pl.ds(c * CHUNK, CHUNK)]],
                         bufs[s], sems[s])

    def drain(s):
        pltpu.make_async_copy(o1.at[pl.ds(0, CHUNK)], bufs[s], sems[s]).wait()

    def writeback(c, s):
        pltpu.sync_copy(bufs[s], o1.at[pl.ds(wbase + c * CHUNK, CHUNK)])

    fire(0, 0)
    fire(1, 1)

    def pair(p, _):
        c = 2 * p
        drain(0)
        writeback(c, 0)
        fire(c + 2, 0)
        drain(1)
        writeback(c + 1, 1)
        fire(c + 3, 1)
        return _

    lax.fori_loop(0, N_CHUNKS // 2 - 1, pair, None)
    drain(0)
    writeback(N_CHUNKS - 2, 0)
    drain(1)
    writeback(N_CHUNKS - 1, 1)


def _sc_gather(ids, id_tab):
    mesh = plsc.VectorSubcoreMesh(core_axis_name="c", subcore_axis_name="s")
    f = pl.kernel(
        _sc_gather_body,
        out_type=jax.ShapeDtypeStruct((N, EMB), jnp.float32),
        mesh=mesh,
        scratch_types=[
            pltpu.VMEM((TOK_PER_W,), jnp.int32),
            pltpu.VMEM((CHUNK, EMB), jnp.float32),
            pltpu.VMEM((CHUNK, EMB), jnp.float32),
            pltpu.SemaphoreType.DMA,
            pltpu.SemaphoreType.DMA,
        ],
    )
    return f(ids, id_tab)


def _tc_body(x1, catb, posb, ct, pt, w1, w2, w3, bb, lw, lb, o):
    cat = catb[0, 0, :].reshape(TB, 1)
    pos = posb[0, 0, :].reshape(TB, 1)
    oh_c = (lax.broadcasted_iota(jnp.int32, (TB, CAT), 1) == cat).astype(jnp.bfloat16)
    oh_p = (lax.broadcasted_iota(jnp.int32, (TB, MAXPOS), 1) == pos).astype(jnp.bfloat16)
    x2 = jnp.dot(oh_c, ct[...],
                 preferred_element_type=jnp.float32).astype(jnp.bfloat16)
    x3 = jnp.dot(oh_p, pt[...],
                 preferred_element_type=jnp.float32).astype(jnp.bfloat16)
    acc = jnp.dot(x1[...].astype(jnp.bfloat16), w1[...],
                  preferred_element_type=jnp.float32)
    acc += jnp.dot(x2, w2[...], preferred_element_type=jnp.float32)
    acc += jnp.dot(x3, w3[...], preferred_element_type=jnp.float32)
    acc += bb[...]
    m = jnp.mean(acc, axis=-1, keepdims=True)
    d = acc - m
    v = jnp.mean(d * d, axis=-1, keepdims=True)
    o[...] = d * lax.rsqrt(v + EPS) * lw[...] + lb[...]


def _tc_linear_ln(x1, cats, poss, cat_tab, pos_tab, Wt, b, ln_w, ln_b):
    Wtb = Wt.astype(jnp.bfloat16)
    grid = (N // TB,)
    xspec = pl.BlockSpec((TB, EMB), lambda i: (i, 0))
    ispec = pl.BlockSpec((1, 1, TB), lambda i: (i, 0, 0))
    wspec = pl.BlockSpec((EMB, HID), lambda i: (0, 0))
    vspec = pl.BlockSpec((1, HID), lambda i: (0, 0))
    return pl.pallas_call(
        _tc_body,
        grid=grid,
        in_specs=[
            xspec, ispec, ispec,
            pl.BlockSpec((CAT, EMB), lambda i: (0, 0)),
            pl.BlockSpec((MAXPOS, EMB), lambda i: (0, 0)),
            wspec, wspec, wspec, vspec, vspec, vspec,
        ],
        out_specs=pl.BlockSpec((TB, HID), lambda i: (i, 0)),
        out_shape=jax.ShapeDtypeStruct((N, HID), jnp.float32),
    )(x1, cats.reshape(N // TB, 1, TB), poss.reshape(N // TB, 1, TB),
      cat_tab.astype(jnp.bfloat16), pos_tab.astype(jnp.bfloat16),
      Wtb[:EMB], Wtb[EMB:2 * EMB], Wtb[2 * EMB:], b.reshape(1, HID),
      ln_w.reshape(1, HID), ln_b.reshape(1, HID))


def kernel(input_ids, category_ids, position_ids, id_table, cat_table,
           pos_table, W, b, ln_w, ln_b):
    ids = input_ids.reshape(-1).astype(jnp.int32)
    cats = category_ids.reshape(-1).astype(jnp.int32)
    poss = position_ids.reshape(-1).astype(jnp.int32)
    e1 = _sc_gather(ids, id_table)
    out = _tc_linear_ln(e1, cats, poss, cat_table, pos_table, W.T, b, ln_w, ln_b)
    return out.reshape(B, S, HID)
